# baseline (speedup 1.0000x reference)
import functools

import jax
import jax.numpy as jnp
from jax import lax
from jax.experimental import pallas as pl
from jax.experimental.pallas import tpu as pltpu
from jax.experimental.pallas import tpu_sc as plsc

E = 8
K = 2
H = 1024
F = 4096
T = 8192
TBLK = 512
NTB = T // TBLK
RBLK = 256
NBLK = T * K // RBLK + E
NPAD = NBLK * RBLK
NC, NS = 2, 16
NW = NC * NS
PREC = jax.lax.Precision.DEFAULT
PREC_EXACT = jax.lax.Precision.HIGHEST


def _router_body(x_ref, wr_ref, logits_ref, meta_ref, counts_ref):
    x = x_ref[...]
    logits = lax.dot_general(x, wr_ref[...], (((1,), (0,)), ((), ())),
                             preferred_element_type=jnp.float32,
                             precision=PREC)
    logits_ref[...] = logits
    lane = lax.broadcasted_iota(jnp.int32, (TBLK, 128), 1)
    valid = lane < E
    neg = jnp.where(valid, logits, -jnp.inf)
    m = jnp.max(neg, axis=1, keepdims=True)
    p = jnp.where(valid, jnp.exp(neg - m), 0.0)
    p = p / jnp.sum(p, axis=1, keepdims=True)
    m1 = jnp.max(p, axis=1, keepdims=True)
    e0 = jnp.min(jnp.where(jnp.logical_and(p == m1, valid), lane, 128),
                 axis=1, keepdims=True)
    p2 = jnp.where(lane == e0, -1.0, p)
    m2 = jnp.max(p2, axis=1, keepdims=True)
    e1 = jnp.min(jnp.where(jnp.logical_and(p2 == m2, valid), lane, 128),
                 axis=1, keepdims=True)
    wsum = m1 + m2
    w0 = m1 / wsum
    w1 = m2 / wsum
    oh0 = jnp.where(lane == e0, 1.0, 0.0)
    oh1 = jnp.where(lane == e1, 1.0, 0.0)
    oh01 = oh0 + oh1
    r = lax.broadcasted_iota(jnp.int32, (TBLK, TBLK), 0)
    c = lax.broadcasted_iota(jnp.int32, (TBLK, TBLK), 1)
    tri = jnp.where(r > c, 1.0, 0.0)
    s = lax.dot_general(tri, oh01, (((1,), (0,)), ((), ())),
                        preferred_element_type=jnp.float32,
                        precision=PREC_EXACT)
    r0 = jnp.sum(oh0 * s, axis=1, keepdims=True)
    r1 = jnp.sum(oh1 * s, axis=1, keepdims=True)
    counts_ref[...] = jnp.sum(oh01, axis=0, keepdims=True).reshape(1, 1, 128)
    meta_ref[...] = jnp.where(lane == 0, e0.astype(jnp.float32),
                    jnp.where(lane == 1, e1.astype(jnp.float32),
                    jnp.where(lane == 2, r0,
                    jnp.where(lane == 3, r1,
                    jnp.where(lane == 4, w0,
                    jnp.where(lane == 5, w1, 0.0))))))


_router = pl.pallas_call(
    _router_body,
    grid=(NTB,),
    in_specs=[
        pl.BlockSpec((TBLK, H), lambda i: (i, 0)),
        pl.BlockSpec((H, 128), lambda i: (0, 0)),
    ],
    out_specs=[
        pl.BlockSpec((TBLK, 128), lambda i: (i, 0)),
        pl.BlockSpec((TBLK, 128), lambda i: (i, 0)),
        pl.BlockSpec((1, 1, 128), lambda i: (i, 0, 0)),
    ],
    out_shape=[
        jax.ShapeDtypeStruct((T, 128), jnp.float32),
        jax.ShapeDtypeStruct((T, 128), jnp.float32),
        jax.ShapeDtypeStruct((NTB, 1, 128), jnp.float32),
    ],
)


def _metadata(meta, counts3):
    counts = counts3.reshape(NTB, 128)[:, :E]
    carry = jnp.cumsum(counts, axis=0) - counts
    total = jnp.sum(counts, axis=0).astype(jnp.int32)
    cap = ((total + RBLK - 1) // RBLK) * RBLK
    ends = jnp.cumsum(cap)
    off = (ends - cap).astype(jnp.float32)
    e0 = meta[:, 0].astype(jnp.int32)
    e1 = meta[:, 1].astype(jnp.int32)
    oh0 = jax.nn.one_hot(e0, E, dtype=jnp.float32)
    oh1 = jax.nn.one_hot(e1, E, dtype=jnp.float32)
    carry_rep = jnp.broadcast_to(carry[:, None, :], (NTB, TBLK, E)).reshape(T, E)
    base = off[None, :] + carry_rep
    d0 = (jnp.sum(oh0 * base, axis=1) + meta[:, 2]).astype(jnp.int32)
    d1 = (jnp.sum(oh1 * base, axis=1) + meta[:, 3]).astype(jnp.int32)
    block_starts = jnp.arange(NBLK, dtype=jnp.int32) * RBLK
    be = jnp.clip(jnp.sum(
        (block_starts[:, None] >= ends[None, :]).astype(jnp.int32), axis=1),
        0, E - 1).astype(jnp.int32)
    return d0, d1, be


_DCH = 64
_TPW = T // NW
_ZCH = 64
_ZPW = K * T // NW


@functools.cache
def _sc_kernels():
    mesh = plsc.VectorSubcoreMesh(core_axis_name="c", subcore_axis_name="s")

    @functools.partial(
        pl.kernel,
        out_type=jax.ShapeDtypeStruct((NPAD, H), jnp.float32),
        mesh=mesh,
        scratch_types=[
            pltpu.VMEM((_DCH,), jnp.int32),
            pltpu.VMEM((_DCH,), jnp.int32),
            pltpu.VMEM((_DCH, H), jnp.float32),
            pltpu.SemaphoreType.DMA,
        ],
    )
    def dispatch(x_hbm, d0_hbm, d1_hbm, xs_hbm, idx0_v, idx1_v, rows_v, sem):
        wid = lax.axis_index("s") * NC + lax.axis_index("c")
        base = wid * _TPW

        def chunk(i, carry):
            tb = base + i * _DCH
            pltpu.sync_copy(d0_hbm.at[pl.ds(tb, _DCH)], idx0_v)
            pltpu.sync_copy(d1_hbm.at[pl.ds(tb, _DCH)], idx1_v)
            pltpu.sync_copy(x_hbm.at[pl.ds(tb, _DCH)], rows_v)
            pltpu.async_copy(rows_v, xs_hbm.at[idx0_v], sem).wait()
            pltpu.async_copy(rows_v, xs_hbm.at[idx1_v], sem).wait()
            return carry

        lax.fori_loop(0, _TPW // _DCH, chunk, 0)

    @functools.partial(
        pl.kernel,
        out_type=jax.ShapeDtypeStruct((K * T, H), jnp.float32),
        mesh=mesh,
        scratch_types=[
            pltpu.VMEM((_ZCH,), jnp.int32),
            pltpu.VMEM((_ZCH, H), jnp.float32),
            pltpu.SemaphoreType.DMA,
        ],
    )
    def unsort(y_hbm, dcat_hbm, z_hbm, idx_v, rows_v, sem):
        wid = lax.axis_index("s") * NC + lax.axis_index("c")
        base = wid * _ZPW

        def chunk(i, carry):
            b = base + i * _ZCH
            pltpu.sync_copy(dcat_hbm.at[pl.ds(b, _ZCH)], idx_v)
            pltpu.async_copy(y_hbm.at[idx_v], rows_v, sem).wait()
            pltpu.sync_copy(rows_v, z_hbm.at[pl.ds(b, _ZCH)])
            return carry

        lax.fori_loop(0, _ZPW // _ZCH, chunk, 0)

    return dispatch, unsort


def _ffn_body(be_ref, xs_ref, wg_ref, wu_ref, wd_ref, y_ref):
    x = xs_ref[...].astype(jnp.bfloat16)
    dn = (((1,), (0,)), ((), ()))
    f2 = F // 2

    def half(c):
        g = lax.dot_general(x, wg_ref[0, :, c * f2:(c + 1) * f2], dn,
                            preferred_element_type=jnp.float32, precision=PREC)
        u = lax.dot_general(x, wu_ref[0, :, c * f2:(c + 1) * f2], dn,
                            preferred_element_type=jnp.float32, precision=PREC)
        hb = (g * jax.nn.sigmoid(g) * u).astype(jnp.bfloat16)
        return lax.dot_general(hb, wd_ref[0, c * f2:(c + 1) * f2, :], dn,
                               preferred_element_type=jnp.float32,
                               precision=PREC)

    y_ref[...] = half(0) + half(1)


_ffn = pl.pallas_call(
    _ffn_body,
    grid_spec=pltpu.PrefetchScalarGridSpec(
        num_scalar_prefetch=1,
        grid=(NBLK,),
        in_specs=[
            pl.BlockSpec((RBLK, H), lambda i, be: (i, 0)),
            pl.BlockSpec((1, H, F), lambda i, be: (be[i], 0, 0)),
            pl.BlockSpec((1, H, F), lambda i, be: (be[i], 0, 0)),
            pl.BlockSpec((1, F, H), lambda i, be: (be[i], 0, 0)),
        ],
        out_specs=pl.BlockSpec((RBLK, H), lambda i, be: (i, 0)),
    ),
    out_shape=jax.ShapeDtypeStruct((NPAD, H), jnp.float32),
    compiler_params=pltpu.CompilerParams(vmem_limit_bytes=100 * 1024 * 1024),
)


def _combine_body(z_ref, meta_ref, out_ref):
    z = z_ref[...]
    out_ref[...] = z[0] * meta_ref[:, 4:5] + z[1] * meta_ref[:, 5:6]


_combine = pl.pallas_call(
    _combine_body,
    grid=(T // RBLK,),
    in_specs=[
        pl.BlockSpec((2, RBLK, H), lambda i: (0, i, 0)),
        pl.BlockSpec((RBLK, 128), lambda i: (i, 0)),
    ],
    out_specs=pl.BlockSpec((RBLK, H), lambda i: (i, 0)),
    out_shape=jax.ShapeDtypeStruct((T, H), jnp.float32),
)


def kernel(hidden_states, Wr, Wg, Wu, Wd):
    b, s, h = hidden_states.shape
    x = hidden_states.reshape(-1, h)
    wr_pad = jnp.pad(Wr, ((0, 0), (0, 128 - E)))
    logits_pad, meta, counts3 = _router(x, wr_pad)
    d0, d1, be = _metadata(meta, counts3)
    dispatch, unsort = _sc_kernels()
    xs = dispatch(x, d0, d1)
    y = _ffn(be, xs, Wg.astype(jnp.bfloat16), Wu.astype(jnp.bfloat16),
             Wd.astype(jnp.bfloat16))
    z = unsort(y, jnp.concatenate([d0, d1])).reshape(K, T, h)
    out = _combine(z, meta)
    return out.reshape(b, s, h), logits_pad[:, :E]


# speedup vs baseline: 1.0816x; 1.0816x over previous
import functools

import jax
import jax.numpy as jnp
from jax import lax
from jax.experimental import pallas as pl
from jax.experimental.pallas import tpu as pltpu
from jax.experimental.pallas import tpu_sc as plsc

E = 8
K = 2
H = 1024
F = 4096
T = 8192
TBLK = 512
NTB = T // TBLK
RBLK = 256
NBLK = T * K // RBLK + E
NPAD = NBLK * RBLK
NC, NS = 2, 16
NW = NC * NS
PREC = jax.lax.Precision.DEFAULT
PREC_EXACT = jax.lax.Precision.HIGHEST


def _router_body(x_ref, wr_ref, logits_ref, meta_ref, counts_ref):
    x = x_ref[...]
    logits = lax.dot_general(x, wr_ref[...], (((1,), (0,)), ((), ())),
                             preferred_element_type=jnp.float32,
                             precision=PREC)
    logits_ref[...] = logits
    lane = lax.broadcasted_iota(jnp.int32, (TBLK, 128), 1)
    valid = lane < E
    neg = jnp.where(valid, logits, -jnp.inf)
    m = jnp.max(neg, axis=1, keepdims=True)
    p = jnp.where(valid, jnp.exp(neg - m), 0.0)
    p = p / jnp.sum(p, axis=1, keepdims=True)
    m1 = jnp.max(p, axis=1, keepdims=True)
    e0 = jnp.min(jnp.where(jnp.logical_and(p == m1, valid), lane, 128),
                 axis=1, keepdims=True)
    p2 = jnp.where(lane == e0, -1.0, p)
    m2 = jnp.max(p2, axis=1, keepdims=True)
    e1 = jnp.min(jnp.where(jnp.logical_and(p2 == m2, valid), lane, 128),
                 axis=1, keepdims=True)
    wsum = m1 + m2
    w0 = m1 / wsum
    w1 = m2 / wsum
    oh0 = jnp.where(lane == e0, 1.0, 0.0)
    oh1 = jnp.where(lane == e1, 1.0, 0.0)
    oh01 = oh0 + oh1
    r = lax.broadcasted_iota(jnp.int32, (TBLK, TBLK), 0)
    c = lax.broadcasted_iota(jnp.int32, (TBLK, TBLK), 1)
    tri = jnp.where(r > c, 1.0, 0.0)
    s = lax.dot_general(tri, oh01, (((1,), (0,)), ((), ())),
                        preferred_element_type=jnp.float32,
                        precision=PREC_EXACT)
    r0 = jnp.sum(oh0 * s, axis=1, keepdims=True)
    r1 = jnp.sum(oh1 * s, axis=1, keepdims=True)
    counts_ref[...] = jnp.sum(oh01, axis=0, keepdims=True).reshape(1, 1, 128)
    meta_ref[...] = jnp.where(lane == 0, e0.astype(jnp.float32),
                    jnp.where(lane == 1, e1.astype(jnp.float32),
                    jnp.where(lane == 2, r0,
                    jnp.where(lane == 3, r1,
                    jnp.where(lane == 4, w0,
                    jnp.where(lane == 5, w1, 0.0))))))


_router = pl.pallas_call(
    _router_body,
    grid=(NTB,),
    in_specs=[
        pl.BlockSpec((TBLK, H), lambda i: (i, 0)),
        pl.BlockSpec((H, 128), lambda i: (0, 0)),
    ],
    out_specs=[
        pl.BlockSpec((TBLK, 128), lambda i: (i, 0)),
        pl.BlockSpec((TBLK, 128), lambda i: (i, 0)),
        pl.BlockSpec((1, 1, 128), lambda i: (i, 0, 0)),
    ],
    out_shape=[
        jax.ShapeDtypeStruct((T, 128), jnp.float32),
        jax.ShapeDtypeStruct((T, 128), jnp.float32),
        jax.ShapeDtypeStruct((NTB, 1, 128), jnp.float32),
    ],
)


def _metadata(meta, counts3):
    counts = counts3.reshape(NTB, 128)[:, :E]
    carry = jnp.cumsum(counts, axis=0) - counts
    total = jnp.sum(counts, axis=0).astype(jnp.int32)
    cap = ((total + RBLK - 1) // RBLK) * RBLK
    ends = jnp.cumsum(cap)
    off = (ends - cap).astype(jnp.float32)
    e0 = meta[:, 0].astype(jnp.int32)
    e1 = meta[:, 1].astype(jnp.int32)
    oh0 = jax.nn.one_hot(e0, E, dtype=jnp.float32)
    oh1 = jax.nn.one_hot(e1, E, dtype=jnp.float32)
    carry_rep = jnp.broadcast_to(carry[:, None, :], (NTB, TBLK, E)).reshape(T, E)
    base = off[None, :] + carry_rep
    d0 = (jnp.sum(oh0 * base, axis=1) + meta[:, 2]).astype(jnp.int32)
    d1 = (jnp.sum(oh1 * base, axis=1) + meta[:, 3]).astype(jnp.int32)
    block_starts = jnp.arange(NBLK, dtype=jnp.int32) * RBLK
    be = jnp.clip(jnp.sum(
        (block_starts[:, None] >= ends[None, :]).astype(jnp.int32), axis=1),
        0, E - 1).astype(jnp.int32)
    nb = (ends[E - 1] // RBLK).reshape(1).astype(jnp.int32)
    return d0, d1, be, nb


_DCH = 64
_TPW = T // NW
_ZCH = 64
_ZPW = K * T // NW


@functools.cache
def _sc_kernels():
    mesh = plsc.VectorSubcoreMesh(core_axis_name="c", subcore_axis_name="s")

    @functools.partial(
        pl.kernel,
        out_type=jax.ShapeDtypeStruct((NPAD, H), jnp.float32),
        mesh=mesh,
        scratch_types=[
            pltpu.VMEM((_DCH,), jnp.int32),
            pltpu.VMEM((_DCH,), jnp.int32),
            pltpu.VMEM((_DCH, H), jnp.float32),
            pltpu.SemaphoreType.DMA,
        ],
    )
    def dispatch(x_hbm, d0_hbm, d1_hbm, xs_hbm, idx0_v, idx1_v, rows_v, sem):
        wid = lax.axis_index("s") * NC + lax.axis_index("c")
        base = wid * _TPW

        def chunk(i, carry):
            tb = base + i * _DCH
            pltpu.sync_copy(d0_hbm.at[pl.ds(tb, _DCH)], idx0_v)
            pltpu.sync_copy(d1_hbm.at[pl.ds(tb, _DCH)], idx1_v)
            pltpu.sync_copy(x_hbm.at[pl.ds(tb, _DCH)], rows_v)
            pltpu.async_copy(rows_v, xs_hbm.at[idx0_v], sem).wait()
            pltpu.async_copy(rows_v, xs_hbm.at[idx1_v], sem).wait()
            return carry

        lax.fori_loop(0, _TPW // _DCH, chunk, 0)

    @functools.partial(
        pl.kernel,
        out_type=jax.ShapeDtypeStruct((K * T, H), jnp.float32),
        mesh=mesh,
        scratch_types=[
            pltpu.VMEM((_ZCH,), jnp.int32),
            pltpu.VMEM((_ZCH, H), jnp.float32),
            pltpu.SemaphoreType.DMA,
        ],
    )
    def unsort(y_hbm, dcat_hbm, z_hbm, idx_v, rows_v, sem):
        wid = lax.axis_index("s") * NC + lax.axis_index("c")
        base = wid * _ZPW

        def chunk(i, carry):
            b = base + i * _ZCH
            pltpu.sync_copy(dcat_hbm.at[pl.ds(b, _ZCH)], idx_v)
            pltpu.async_copy(y_hbm.at[idx_v], rows_v, sem).wait()
            pltpu.sync_copy(rows_v, z_hbm.at[pl.ds(b, _ZCH)])
            return carry

        lax.fori_loop(0, _ZPW // _ZCH, chunk, 0)

    return dispatch, unsort


def _mm1_body(be_ref, nb_ref, xs_ref, wg_ref, wu_ref, h_ref):
    dn = (((1,), (0,)), ((), ()))

    @pl.when(pl.program_id(1) < nb_ref[0])
    def _():
        x = xs_ref[...]
        g = lax.dot_general(x, wg_ref[0], dn,
                            preferred_element_type=jnp.float32, precision=PREC)
        u = lax.dot_general(x, wu_ref[0], dn,
                            preferred_element_type=jnp.float32, precision=PREC)
        h_ref[...] = (g * jax.nn.sigmoid(g) * u).astype(jnp.bfloat16)


_mm1 = pl.pallas_call(
    _mm1_body,
    grid_spec=pltpu.PrefetchScalarGridSpec(
        num_scalar_prefetch=2,
        grid=(2, NBLK),
        in_specs=[
            pl.BlockSpec((RBLK, H), lambda j, i, be, nb: (i, 0)),
            pl.BlockSpec((1, H, F // 2), lambda j, i, be, nb: (be[i], 0, j)),
            pl.BlockSpec((1, H, F // 2), lambda j, i, be, nb: (be[i], 0, j)),
        ],
        out_specs=pl.BlockSpec((RBLK, F // 2), lambda j, i, be, nb: (i, j)),
    ),
    out_shape=jax.ShapeDtypeStruct((NPAD, F), jnp.bfloat16),
)


def _mm2_body(be_ref, nb_ref, h_ref, wd_ref, y_ref):
    dn = (((1,), (0,)), ((), ()))

    @pl.when(pl.program_id(0) < nb_ref[0])
    def _():
        y_ref[...] = lax.dot_general(h_ref[...], wd_ref[0], dn,
                                     preferred_element_type=jnp.float32,
                                     precision=PREC)


_mm2 = pl.pallas_call(
    _mm2_body,
    grid_spec=pltpu.PrefetchScalarGridSpec(
        num_scalar_prefetch=2,
        grid=(NBLK,),
        in_specs=[
            pl.BlockSpec((RBLK, F), lambda i, be, nb: (i, 0)),
            pl.BlockSpec((1, F, H), lambda i, be, nb: (be[i], 0, 0)),
        ],
        out_specs=pl.BlockSpec((RBLK, H), lambda i, be, nb: (i, 0)),
    ),
    out_shape=jax.ShapeDtypeStruct((NPAD, H), jnp.float32),
)


def _combine_body(z_ref, meta_ref, out_ref):
    z = z_ref[...]
    out_ref[...] = z[0] * meta_ref[:, 4:5] + z[1] * meta_ref[:, 5:6]


_combine = pl.pallas_call(
    _combine_body,
    grid=(T // RBLK,),
    in_specs=[
        pl.BlockSpec((2, RBLK, H), lambda i: (0, i, 0)),
        pl.BlockSpec((RBLK, 128), lambda i: (i, 0)),
    ],
    out_specs=pl.BlockSpec((RBLK, H), lambda i: (i, 0)),
    out_shape=jax.ShapeDtypeStruct((T, H), jnp.float32),
)


def kernel(hidden_states, Wr, Wg, Wu, Wd):
    b, s, h = hidden_states.shape
    x = hidden_states.reshape(-1, h)
    wr_pad = jnp.pad(Wr, ((0, 0), (0, 128 - E)))
    logits_pad, meta, counts3 = _router(x, wr_pad)
    d0, d1, be, nb = _metadata(meta, counts3)
    dispatch, unsort = _sc_kernels()
    xs = dispatch(x, d0, d1)
    hmid = _mm1(be, nb, xs, Wg, Wu)
    y = _mm2(be, nb, hmid, Wd)
    z = unsort(y, jnp.concatenate([d0, d1])).reshape(K, T, h)
    out = _combine(z, meta)
    return out.reshape(b, s, h), logits_pad[:, :E]


# speedup vs baseline: 1.1108x; 1.0270x over previous
import functools

import jax
import jax.numpy as jnp
from jax import lax
from jax.experimental import pallas as pl
from jax.experimental.pallas import tpu as pltpu
from jax.experimental.pallas import tpu_sc as plsc

E = 8
K = 2
H = 1024
F = 4096
T = 8192
TBLK = 512
NTB = T // TBLK
RBLK = 512
NBLK = T * K // RBLK + E
NPAD = NBLK * RBLK
NC, NS = 2, 16
NW = NC * NS
PREC = jax.lax.Precision.DEFAULT
PREC_EXACT = jax.lax.Precision.HIGHEST


def _router_body(x_ref, wr_ref, logits_ref, meta_ref, counts_ref):
    x = x_ref[...]
    logits = lax.dot_general(x, wr_ref[...], (((1,), (0,)), ((), ())),
                             preferred_element_type=jnp.float32,
                             precision=PREC)
    logits_ref[...] = logits
    lane = lax.broadcasted_iota(jnp.int32, (TBLK, 128), 1)
    valid = lane < E
    neg = jnp.where(valid, logits, -jnp.inf)
    m = jnp.max(neg, axis=1, keepdims=True)
    p = jnp.where(valid, jnp.exp(neg - m), 0.0)
    p = p / jnp.sum(p, axis=1, keepdims=True)
    m1 = jnp.max(p, axis=1, keepdims=True)
    e0 = jnp.min(jnp.where(jnp.logical_and(p == m1, valid), lane, 128),
                 axis=1, keepdims=True)
    p2 = jnp.where(lane == e0, -1.0, p)
    m2 = jnp.max(p2, axis=1, keepdims=True)
    e1 = jnp.min(jnp.where(jnp.logical_and(p2 == m2, valid), lane, 128),
                 axis=1, keepdims=True)
    wsum = m1 + m2
    w0 = m1 / wsum
    w1 = m2 / wsum
    oh0 = jnp.where(lane == e0, 1.0, 0.0)
    oh1 = jnp.where(lane == e1, 1.0, 0.0)
    oh01 = oh0 + oh1
    r = lax.broadcasted_iota(jnp.int32, (TBLK, TBLK), 0)
    c = lax.broadcasted_iota(jnp.int32, (TBLK, TBLK), 1)
    tri = jnp.where(r > c, 1.0, 0.0)
    s = lax.dot_general(tri, oh01, (((1,), (0,)), ((), ())),
                        preferred_element_type=jnp.float32,
                        precision=PREC_EXACT)
    r0 = jnp.sum(oh0 * s, axis=1, keepdims=True)
    r1 = jnp.sum(oh1 * s, axis=1, keepdims=True)
    counts_ref[...] = jnp.sum(oh01, axis=0, keepdims=True).reshape(1, 1, 128)
    meta_ref[...] = jnp.where(lane == 0, e0.astype(jnp.float32),
                    jnp.where(lane == 1, e1.astype(jnp.float32),
                    jnp.where(lane == 2, r0,
                    jnp.where(lane == 3, r1,
                    jnp.where(lane == 4, w0,
                    jnp.where(lane == 5, w1, 0.0))))))


_router = pl.pallas_call(
    _router_body,
    grid=(NTB,),
    in_specs=[
        pl.BlockSpec((TBLK, H), lambda i: (i, 0)),
        pl.BlockSpec((H, 128), lambda i: (0, 0)),
    ],
    out_specs=[
        pl.BlockSpec((TBLK, 128), lambda i: (i, 0)),
        pl.BlockSpec((TBLK, 128), lambda i: (i, 0)),
        pl.BlockSpec((1, 1, 128), lambda i: (i, 0, 0)),
    ],
    out_shape=[
        jax.ShapeDtypeStruct((T, 128), jnp.float32),
        jax.ShapeDtypeStruct((T, 128), jnp.float32),
        jax.ShapeDtypeStruct((NTB, 1, 128), jnp.float32),
    ],
)


def _metadata(meta, counts3):
    counts = counts3.reshape(NTB, 128)[:, :E]
    carry = jnp.cumsum(counts, axis=0) - counts
    total = jnp.sum(counts, axis=0).astype(jnp.int32)
    cap = ((total + RBLK - 1) // RBLK) * RBLK
    ends = jnp.cumsum(cap)
    off = (ends - cap).astype(jnp.float32)
    e0 = meta[:, 0].astype(jnp.int32)
    e1 = meta[:, 1].astype(jnp.int32)
    oh0 = jax.nn.one_hot(e0, E, dtype=jnp.float32)
    oh1 = jax.nn.one_hot(e1, E, dtype=jnp.float32)
    carry_rep = jnp.broadcast_to(carry[:, None, :], (NTB, TBLK, E)).reshape(T, E)
    base = off[None, :] + carry_rep
    d0 = (jnp.sum(oh0 * base, axis=1) + meta[:, 2]).astype(jnp.int32)
    d1 = (jnp.sum(oh1 * base, axis=1) + meta[:, 3]).astype(jnp.int32)
    block_starts = jnp.arange(NBLK, dtype=jnp.int32) * RBLK
    be = jnp.clip(jnp.sum(
        (block_starts[:, None] >= ends[None, :]).astype(jnp.int32), axis=1),
        0, E - 1).astype(jnp.int32)
    nb = (ends[E - 1] // RBLK).reshape(1).astype(jnp.int32)
    return d0, d1, be, nb


_DCH = 64
_TPW = T // NW
_ZCH = 64
_ZPW = K * T // NW


@functools.cache
def _sc_kernels():
    mesh = plsc.VectorSubcoreMesh(core_axis_name="c", subcore_axis_name="s")

    @functools.partial(
        pl.kernel,
        out_type=jax.ShapeDtypeStruct((NPAD, H), jnp.float32),
        mesh=mesh,
        scratch_types=[
            pltpu.VMEM((_DCH,), jnp.int32),
            pltpu.VMEM((_DCH,), jnp.int32),
            pltpu.VMEM((_DCH, H), jnp.float32),
            pltpu.SemaphoreType.DMA,
        ],
    )
    def dispatch(x_hbm, d0_hbm, d1_hbm, xs_hbm, idx0_v, idx1_v, rows_v, sem):
        wid = lax.axis_index("s") * NC + lax.axis_index("c")
        base = wid * _TPW

        def chunk(i, carry):
            tb = base + i * _DCH
            pltpu.sync_copy(d0_hbm.at[pl.ds(tb, _DCH)], idx0_v)
            pltpu.sync_copy(d1_hbm.at[pl.ds(tb, _DCH)], idx1_v)
            pltpu.sync_copy(x_hbm.at[pl.ds(tb, _DCH)], rows_v)
            pltpu.async_copy(rows_v, xs_hbm.at[idx0_v], sem).wait()
            pltpu.async_copy(rows_v, xs_hbm.at[idx1_v], sem).wait()
            return carry

        lax.fori_loop(0, _TPW // _DCH, chunk, 0)

    @functools.partial(
        pl.kernel,
        out_type=jax.ShapeDtypeStruct((K * T, H), jnp.float32),
        mesh=mesh,
        scratch_types=[
            pltpu.VMEM((_ZCH,), jnp.int32),
            pltpu.VMEM((_ZCH, H), jnp.float32),
            pltpu.SemaphoreType.DMA,
        ],
    )
    def unsort(y_hbm, dcat_hbm, z_hbm, idx_v, rows_v, sem):
        wid = lax.axis_index("s") * NC + lax.axis_index("c")
        base = wid * _ZPW

        def chunk(i, carry):
            b = base + i * _ZCH
            pltpu.sync_copy(dcat_hbm.at[pl.ds(b, _ZCH)], idx_v)
            pltpu.async_copy(y_hbm.at[idx_v], rows_v, sem).wait()
            pltpu.sync_copy(rows_v, z_hbm.at[pl.ds(b, _ZCH)])
            return carry

        lax.fori_loop(0, _ZPW // _ZCH, chunk, 0)

    return dispatch, unsort


def _mm1_body(be_ref, nb_ref, xs_ref, wg_ref, wu_ref, h_ref):
    dn = (((1,), (0,)), ((), ()))

    @pl.when(pl.program_id(1) < nb_ref[0])
    def _():
        x = xs_ref[...]
        g = lax.dot_general(x, wg_ref[0], dn,
                            preferred_element_type=jnp.float32, precision=PREC)
        u = lax.dot_general(x, wu_ref[0], dn,
                            preferred_element_type=jnp.float32, precision=PREC)
        h_ref[...] = (g * jax.nn.sigmoid(g) * u).astype(jnp.bfloat16)


_mm1 = pl.pallas_call(
    _mm1_body,
    grid_spec=pltpu.PrefetchScalarGridSpec(
        num_scalar_prefetch=2,
        grid=(2, NBLK),
        in_specs=[
            pl.BlockSpec((RBLK, H), lambda j, i, be, nb: (i, 0)),
            pl.BlockSpec((1, H, F // 2), lambda j, i, be, nb: (be[i], 0, j)),
            pl.BlockSpec((1, H, F // 2), lambda j, i, be, nb: (be[i], 0, j)),
        ],
        out_specs=pl.BlockSpec((RBLK, F // 2), lambda j, i, be, nb: (i, j)),
    ),
    out_shape=jax.ShapeDtypeStruct((NPAD, F), jnp.bfloat16),
)


def _mm2_body(be_ref, nb_ref, h_ref, wd_ref, y_ref):
    dn = (((1,), (0,)), ((), ()))

    @pl.when(pl.program_id(0) < nb_ref[0])
    def _():
        y_ref[...] = lax.dot_general(h_ref[...], wd_ref[0], dn,
                                     preferred_element_type=jnp.float32,
                                     precision=PREC)


_mm2 = pl.pallas_call(
    _mm2_body,
    grid_spec=pltpu.PrefetchScalarGridSpec(
        num_scalar_prefetch=2,
        grid=(NBLK,),
        in_specs=[
            pl.BlockSpec((RBLK, F), lambda i, be, nb: (i, 0)),
            pl.BlockSpec((1, F, H), lambda i, be, nb: (be[i], 0, 0)),
        ],
        out_specs=pl.BlockSpec((RBLK, H), lambda i, be, nb: (i, 0)),
    ),
    out_shape=jax.ShapeDtypeStruct((NPAD, H), jnp.float32),
)


def _combine_body(z_ref, meta_ref, out_ref):
    z = z_ref[...]
    out_ref[...] = z[0] * meta_ref[:, 4:5] + z[1] * meta_ref[:, 5:6]


_combine = pl.pallas_call(
    _combine_body,
    grid=(T // RBLK,),
    in_specs=[
        pl.BlockSpec((2, RBLK, H), lambda i: (0, i, 0)),
        pl.BlockSpec((RBLK, 128), lambda i: (i, 0)),
    ],
    out_specs=pl.BlockSpec((RBLK, H), lambda i: (i, 0)),
    out_shape=jax.ShapeDtypeStruct((T, H), jnp.float32),
)


def kernel(hidden_states, Wr, Wg, Wu, Wd):
    b, s, h = hidden_states.shape
    x = hidden_states.reshape(-1, h)
    wr_pad = jnp.pad(Wr, ((0, 0), (0, 128 - E)))
    logits_pad, meta, counts3 = _router(x, wr_pad)
    d0, d1, be, nb = _metadata(meta, counts3)
    dispatch, unsort = _sc_kernels()
    xs = dispatch(x, d0, d1)
    hmid = _mm1(be, nb, xs, Wg, Wu)
    y = _mm2(be, nb, hmid, Wd)
    z = unsort(y, jnp.concatenate([d0, d1])).reshape(K, T, h)
    out = _combine(z, meta)
    return out.reshape(b, s, h), logits_pad[:, :E]
